# baseline (speedup 1.0000x reference)
import functools

import jax
import jax.numpy as jnp
import jax.random as jr
from jax import lax
from jax.experimental import pallas as pl
from jax.experimental.pallas import tpu as pltpu
from jax.experimental.pallas import tpu_sc as plsc

G = 1024
NA = 8
L = 16
ROWS_PER_TILE = G // 16
SUB = 2
SUB_ROWS = ROWS_PER_TILE // SUB
APPLE = 1.0
ORCHARD = 2.0


def _subcore_id():
    return lax.axis_index("s")


_U = jnp.uint32


def _rotl(x, d):
    return (x << _U(d)) | (x >> _U(32 - d))


def _tf2x32(k1, k2, x0, x1):
    ks2 = k1 ^ k2 ^ _U(0x1BD11BDA)
    ks = (k1, k2, ks2)
    rot1 = (13, 15, 26, 6)
    rot2 = (17, 29, 16, 24)
    sched = ((1, 2, 1), (2, 0, 2), (0, 1, 3), (1, 2, 4), (2, 0, 5))
    rots = (rot1, rot2, rot1, rot2, rot1)
    x0 = x0 + k1
    x1 = x1 + k2
    for g in range(5):
        for r in rots[g]:
            x0 = x0 + x1
            x1 = _rotl(x1, r)
            x1 = x1 ^ x0
        a, b, c = sched[g]
        x0 = x0 + ks[a]
        x1 = x1 + ks[b] + _U(c)
    return x0, x1


def _tf_split(k1, k2):
    a0, b0 = _tf2x32(k1, k2, _U(0), _U(0))
    a1, b1 = _tf2x32(k1, k2, _U(0), _U(1))
    return (a0, b0), (a1, b1)


def _tf_bits32(k1, k2):
    a, b = _tf2x32(k1, k2, _U(0), _U(0))
    return a ^ b


def _body(grid_in, scal_in, grid_out, obs_out, pos_out,
          buf, cnts_l, ttl_l, cnts_sh, ttl_sh, cnts_v, ttl_v, rowb, scal_v,
          rows16, rid_v, p_v, obs_v,
          sem_i0, sem_i1, sem_i2, sem_i3, sem_o, sem2):
    w = _subcore_id()
    base = w * ROWS_PER_TILE
    lanes = lax.iota(jnp.int32, L)

    @pl.when(w == 0)
    def _prefetch():
        pltpu.sync_copy(scal_in, scal_v)

    sems_in = (sem_i0, sem_i1)
    cps_in = [
        pltpu.async_copy(grid_in.at[pl.ds(base + c * SUB_ROWS, SUB_ROWS)],
                         buf.at[pl.ds(c * SUB_ROWS, SUB_ROWS)], sems_in[c])
        for c in range(SUB)
    ]
    cps_out = []
    tvec = jnp.zeros(L, jnp.int32)
    for c in range(SUB):
        cps_in[c].wait()
        cps_out.append(
            pltpu.async_copy(buf.at[pl.ds(c * SUB_ROWS, SUB_ROWS)],
                             grid_out.at[pl.ds(base + c * SUB_ROWS, SUB_ROWS)],
                             sem_o))

        def row_body(r16, cvec, _c=c):
            r = _c * SUB_ROWS + r16

            def chunk_body(j, accs):
                accs = list(accs)
                for u in range(8):
                    v = buf[r, pl.ds((j * 8 + u) * L, L)]
                    accs[u % 4] = accs[u % 4] + jnp.where(v == ORCHARD, 1, 0)
                return tuple(accs)

            z = jnp.zeros(L, jnp.int32)
            a0, a1, a2, a3 = lax.fori_loop(0, G // L // 8, chunk_body,
                                           (z, z, z, z))
            acc = (a0 + a1) + (a2 + a3)
            return jnp.where(lanes == r16, jnp.sum(acc), cvec)

        cvec = lax.fori_loop(0, SUB_ROWS, row_body, jnp.zeros(L, jnp.int32))
        cnts_l[pl.ds(c * SUB_ROWS, L)] = cvec
        tvec = tvec + cvec

    ttl_l[...] = jnp.broadcast_to(jnp.sum(tvec), (L,))
    pltpu.sync_copy(cnts_l, cnts_sh.at[pl.ds(base, ROWS_PER_TILE)])
    pltpu.sync_copy(ttl_l, ttl_sh.at[pl.ds(w * L, L)])
    for c in range(SUB):
        cps_out[c].wait()
    plsc.subcore_barrier()

    @pl.when(w == 0)
    def _tile0():
        pltpu.sync_copy(cnts_sh, cnts_v)
        pltpu.sync_copy(ttl_sh, ttl_v)
        prm = scal_v[pl.ds(3 * L, L)]
        eatv = scal_v[pl.ds(2 * L, L)]

        totals = plsc.load_gather(ttl_v, [lanes * L])
        tot = jnp.sum(totals)

        hi = lax.bitcast_convert_type(prm[1], jnp.uint32)
        lo = lax.bitcast_convert_type(prm[2], jnp.uint32)
        span = jnp.where(tot <= 0, jnp.uint32(1), tot.astype(jnp.uint32))
        mult = jnp.uint32(2 ** 16) % span
        mult = (mult * mult) % span
        ridx = (((hi % span) * mult + (lo % span)) % span).astype(jnp.int32)

        tcs = jnp.cumsum(totals)
        tlane = jnp.min(jnp.where(tcs > ridx, lanes, L))
        tcs_at = jnp.sum(jnp.where(lanes == tlane, tcs, 0))
        t_at = jnp.sum(jnp.where(lanes == tlane, totals, 0))
        rem0 = ridx - (tcs_at - t_at)

        def loc_row(j, carry):
            found, row, k, rem = carry
            c = cnts_v[pl.ds(tlane * ROWS_PER_TILE + j * L, L)]
            tot_j = jnp.sum(c)
            cs = jnp.cumsum(c)
            hit = (found == 0) & (tot_j > rem)
            lane = jnp.min(jnp.where(cs > rem, lanes, L))
            cs_at = jnp.sum(jnp.where(lanes == lane, cs, 0))
            c_at = jnp.sum(jnp.where(lanes == lane, c, 0))
            row2 = jnp.where(hit, tlane * ROWS_PER_TILE + j * L + lane, row)
            k2 = jnp.where(hit, rem - (cs_at - c_at), k)
            rem2 = jnp.where((found == 0) & jnp.logical_not(hit), rem - tot_j, rem)
            return (jnp.where(hit, 1, found), row2, k2, rem2)

        _, row, k, _ = lax.fori_loop(0, ROWS_PER_TILE // L, loc_row,
                                     (jnp.int32(0), jnp.int32(0), jnp.int32(0),
                                      rem0))

        pltpu.sync_copy(grid_in.at[row], rowb)

        def loc_col(j, carry):
            found, col, rem = carry
            c = jnp.where(rowb[pl.ds(j * L, L)] == ORCHARD, 1, 0)
            tot_j = jnp.sum(c)
            cs = jnp.cumsum(c)
            hit = (found == 0) & (tot_j > rem)
            lane = jnp.min(jnp.where(cs > rem, lanes, L))
            col2 = jnp.where(hit, j * L + lane, col)
            rem2 = jnp.where((found == 0) & jnp.logical_not(hit), rem - tot_j, rem)
            return (jnp.where(hit, 1, found), col2, rem2)

        _, col, _ = lax.fori_loop(0, G // L, loc_col,
                                  (jnp.int32(0), jnp.int32(0), k))

        spawn_on = (prm[0] > 0) & (tot > 0)
        spawnP = row * G + col

        pos = scal_v[pl.ds(0, L)]
        mov = scal_v[pl.ds(L, L)]
        x = pos >> 10
        y = pos & (G - 1)
        a = jnp.clip(mov, 0, 4)
        dx = jnp.where(a == 1, -1, 0) + jnp.where(a == 2, 1, 0)
        dy = jnp.where(a == 3, -1, 0) + jnp.where(a == 4, 1, 0)
        nx = jnp.clip(x + dx, 0, G - 1)
        ny = jnp.clip(y + dy, 0, G - 1)
        P = nx * G + ny
        p_v[...] = P

        nx0 = nx[0]
        srow = jnp.where(spawn_on, row, nx0)
        rid = jnp.where(lanes < NA, nx, jnp.where(lanes == NA, srow, nx0))
        rid_v[...] = rid
        pltpu.async_copy(grid_in.at[rid_v], rows16, sem2).wait()
        O = plsc.load_gather(rows16, [lanes, ny])
        base_vals = jnp.where(spawn_on & (P == spawnP), APPLE, O)

        ate_vec = jnp.zeros(L, jnp.int32)
        ate_s = [None] * NA
        for i in reversed(range(NA)):
            prior = jnp.sum(jnp.where((ate_vec > 0) & (P == P[i]), 1, 0))
            cur = jnp.where(prior > 0, ORCHARD, base_vals[i])
            ate_i = (eatv[i] >= 1) & (cur == APPLE)
            ate_s[i] = ate_i
            ate_vec = jnp.where((lanes == i) & ate_i, 1, ate_vec)

        eaten_at = jnp.zeros(L, jnp.int32)
        for j in range(NA):
            eaten_at = eaten_at | jnp.where((P == P[j]) & ate_s[j], 1, 0)

        vfin = jnp.where(eaten_at > 0, ORCHARD, base_vals)
        item = jnp.where(nx == 1, 0.0, jnp.where(vfin == APPLE, 1.0, 2.0))
        obs_v[pl.ds(0, L)] = P.astype(jnp.float32)
        obs_v[pl.ds(L, L)] = item
        obs_v[pl.ds(2 * L, L)] = ate_vec.astype(jnp.float32)
        obs_v[pl.ds(3 * L, L)] = jnp.zeros(L, jnp.float32)
        pltpu.sync_copy(obs_v, obs_out)
        pltpu.sync_copy(p_v, pos_out)

        ate_b = ate_vec > 0
        ones_f = jnp.full(L, APPLE, jnp.float32)
        twos_f = jnp.full(L, ORCHARD, jnp.float32)
        col_vec = jnp.full(L, col, jnp.int32)

        def patch_slot(s, _):
            rs = jnp.sum(jnp.where(lanes == s, rid, 0))
            s_vec = jnp.full(L, 1, jnp.int32) * s
            plsc.store_scatter(rows16, [s_vec, col_vec], ones_f,
                               mask=(lanes == 0) & spawn_on & (row == rs))
            plsc.store_scatter(rows16, [s_vec, ny], twos_f,
                               mask=(lanes < NA) & ate_b & (nx == rs))
            return 0

        lax.fori_loop(0, L, patch_slot, 0)
        pltpu.async_copy(rows16, grid_out.at[rid_v], sem2).wait()


@functools.partial(jax.jit, static_argnames=())
def kernel(grid, agent_position, actions, rng_key):
    kd = jr.key_data(rng_key).astype(jnp.uint32)
    _, ske = _tf_split(kd[0], kd[1])
    r1m, skm = _tf_split(*ske)
    dec_bits = _tf_bits32(*skm)
    dec = lax.bitcast_convert_type(
        (dec_bits >> _U(9)) | _U(0x3F800000), jnp.float32) - jnp.float32(1.0)
    _, sk2 = _tf_split(*r1m)
    ka, kb = _tf_split(*sk2)
    hi = lax.bitcast_convert_type(_tf_bits32(*ka), jnp.int32)
    lo = lax.bitcast_convert_type(_tf_bits32(*kb), jnp.int32)
    sf = (dec < 0.5).astype(jnp.int32)

    z8 = jnp.zeros(NA, jnp.int32)
    scal = jnp.concatenate([
        agent_position[0].astype(jnp.int32), z8,
        actions[:, 0].astype(jnp.int32), z8,
        actions[:, 1].astype(jnp.int32), z8,
        jnp.stack([sf, hi, lo]), jnp.zeros(13, jnp.int32),
    ])

    mesh = plsc.VectorSubcoreMesh(core_axis_name="c", subcore_axis_name="s",
                                  num_cores=1, num_subcores=16)
    grid_out, obs, poso = pl.kernel(
        _body,
        out_type=[
            jax.ShapeDtypeStruct((G, G), jnp.float32),
            jax.ShapeDtypeStruct((4 * L,), jnp.float32),
            jax.ShapeDtypeStruct((L,), jnp.int32),
        ],
        mesh=mesh,
        compiler_params=pltpu.CompilerParams(needs_layout_passes=False),
        scratch_types=[
            pltpu.VMEM((ROWS_PER_TILE, G), jnp.float32),
            pltpu.VMEM((ROWS_PER_TILE,), jnp.int32),
            pltpu.VMEM((L,), jnp.int32),
            pltpu.VMEM_SHARED((G,), jnp.int32),
            pltpu.VMEM_SHARED((L * L,), jnp.int32),
            pltpu.VMEM((G,), jnp.int32),
            pltpu.VMEM((L * L,), jnp.int32),
            pltpu.VMEM((G,), jnp.float32),
            pltpu.VMEM((4 * L,), jnp.int32),
            pltpu.VMEM((L, G), jnp.float32),
            pltpu.VMEM((L,), jnp.int32),
            pltpu.VMEM((L,), jnp.int32),
            pltpu.VMEM((4 * L,), jnp.float32),
            pltpu.SemaphoreType.DMA,
            pltpu.SemaphoreType.DMA,
            pltpu.SemaphoreType.DMA,
            pltpu.SemaphoreType.DMA,
            pltpu.SemaphoreType.DMA,
            pltpu.SemaphoreType.DMA,
        ],
    )(grid[0], scal)

    loc_obs = obs[0:NA].reshape(NA, 1)
    item_obs = obs[L:L + NA].reshape(NA, 1)
    rew_obs = obs[2 * L:2 * L + NA].reshape(NA, 1)
    return (loc_obs, item_obs, rew_obs, grid_out[None],
            poso[0:NA].reshape(1, NA))


# speedup vs baseline: 1.1140x; 1.1140x over previous
import functools

import jax
import jax.numpy as jnp
import jax.random as jr
from jax import lax
from jax.experimental import pallas as pl
from jax.experimental.pallas import tpu as pltpu
from jax.experimental.pallas import tpu_sc as plsc

G = 1024
NA = 8
L = 16
ROWS_PER_TILE = G // 16
SUB = 4
SUB_ROWS = ROWS_PER_TILE // SUB
APPLE = 1.0
ORCHARD = 2.0


def _subcore_id():
    return lax.axis_index("s")


_U = jnp.uint32


def _rotl(x, d):
    return (x << _U(d)) | (x >> _U(32 - d))


def _tf2x32(k1, k2, x0, x1):
    ks2 = k1 ^ k2 ^ _U(0x1BD11BDA)
    ks = (k1, k2, ks2)
    rot1 = (13, 15, 26, 6)
    rot2 = (17, 29, 16, 24)
    sched = ((1, 2, 1), (2, 0, 2), (0, 1, 3), (1, 2, 4), (2, 0, 5))
    rots = (rot1, rot2, rot1, rot2, rot1)
    x0 = x0 + k1
    x1 = x1 + k2
    for g in range(5):
        for r in rots[g]:
            x0 = x0 + x1
            x1 = _rotl(x1, r)
            x1 = x1 ^ x0
        a, b, c = sched[g]
        x0 = x0 + ks[a]
        x1 = x1 + ks[b] + _U(c)
    return x0, x1


def _tf_split(k1, k2):
    a0, b0 = _tf2x32(k1, k2, _U(0), _U(0))
    a1, b1 = _tf2x32(k1, k2, _U(0), _U(1))
    return (a0, b0), (a1, b1)


def _tf_bits32(k1, k2):
    a, b = _tf2x32(k1, k2, _U(0), _U(0))
    return a ^ b


def _body(grid_in, scal_in, act_in, grid_out, obs_out, pos_out,
          buf, cnts_l, ttl_l, cnts_sh, ttl_sh, cnts_v, ttl_v, rowb, scal_v,
          act_v, rows16, rid_v, p_v, obs_v,
          sem_i0, sem_i1, sem_i2, sem_i3, sem_o, sem2):
    w = _subcore_id()
    base = w * ROWS_PER_TILE
    lanes = lax.iota(jnp.int32, L)

    @pl.when(w == 0)
    def _prefetch():
        pltpu.sync_copy(scal_in, scal_v)
        pltpu.sync_copy(act_in, act_v)

    sems_in = (sem_i0, sem_i1, sem_i2, sem_i3)
    cps_in = [
        pltpu.async_copy(grid_in.at[pl.ds(base + c * SUB_ROWS, SUB_ROWS)],
                         buf.at[pl.ds(c * SUB_ROWS, SUB_ROWS)], sems_in[c])
        for c in range(SUB)
    ]
    cps_out = []
    tvec = jnp.zeros(L, jnp.int32)
    for c in range(SUB):
        cps_in[c].wait()
        cps_out.append(
            pltpu.async_copy(buf.at[pl.ds(c * SUB_ROWS, SUB_ROWS)],
                             grid_out.at[pl.ds(base + c * SUB_ROWS, SUB_ROWS)],
                             sem_o))

        def row_body(r16, cvec, _c=c):
            r = _c * SUB_ROWS + r16

            def chunk_body(j, acc):
                for u in range(8):
                    v = buf[r, pl.ds((j * 8 + u) * L, L)]
                    acc = acc + jnp.where(v == ORCHARD, 1, 0)
                return acc

            acc = lax.fori_loop(0, G // L // 8, chunk_body,
                                jnp.zeros(L, jnp.int32))
            return jnp.where(lanes == r16, jnp.sum(acc), cvec)

        cvec = lax.fori_loop(0, SUB_ROWS, row_body, jnp.zeros(L, jnp.int32))
        cnts_l[pl.ds(c * SUB_ROWS, L)] = cvec
        tvec = tvec + cvec

    ttl_l[...] = jnp.broadcast_to(jnp.sum(tvec), (L,))
    pltpu.sync_copy(cnts_l, cnts_sh.at[pl.ds(base, ROWS_PER_TILE)])
    pltpu.sync_copy(ttl_l, ttl_sh.at[pl.ds(w * L, L)])
    for c in range(SUB):
        cps_out[c].wait()
    plsc.subcore_barrier()

    @pl.when(w == 0)
    def _tile0():
        pltpu.sync_copy(cnts_sh, cnts_v)
        pltpu.sync_copy(ttl_sh, ttl_v)
        sv = scal_v[pl.ds(0, L)]
        eatv = plsc.load_gather(act_v, [(lanes * 2 + 1) & (L - 1)])
        mov = plsc.load_gather(act_v, [(lanes * 2) & (L - 1)])

        k1 = lax.bitcast_convert_type(sv[NA], jnp.uint32)
        k2 = lax.bitcast_convert_type(sv[NA + 1], jnp.uint32)
        _, ske = _tf_split(k1, k2)
        r1m, skm = _tf_split(*ske)
        dec_bits = _tf_bits32(*skm)
        dec = lax.bitcast_convert_type(
            (dec_bits >> _U(9)) | _U(0x3F800000), jnp.float32) - jnp.float32(1.0)
        sf = dec < jnp.float32(0.5)
        _, sk2 = _tf_split(*r1m)
        ka, kb = _tf_split(*sk2)
        hi = _tf_bits32(*ka)
        lo = _tf_bits32(*kb)

        totals = plsc.load_gather(ttl_v, [lanes * L])
        tot = jnp.sum(totals)

        span = jnp.where(tot <= 0, jnp.uint32(1), tot.astype(jnp.uint32))
        mult = jnp.uint32(2 ** 16) % span
        mult = (mult * mult) % span
        ridx = (((hi % span) * mult + (lo % span)) % span).astype(jnp.int32)

        tcs = jnp.cumsum(totals)
        tlane = jnp.min(jnp.where(tcs > ridx, lanes, L))
        tcs_at = jnp.sum(jnp.where(lanes == tlane, tcs, 0))
        t_at = jnp.sum(jnp.where(lanes == tlane, totals, 0))
        rem0 = ridx - (tcs_at - t_at)

        def loc_row(j, carry):
            found, row, k, rem = carry
            c = cnts_v[pl.ds(tlane * ROWS_PER_TILE + j * L, L)]
            tot_j = jnp.sum(c)
            cs = jnp.cumsum(c)
            hit = (found == 0) & (tot_j > rem)
            lane = jnp.min(jnp.where(cs > rem, lanes, L))
            cs_at = jnp.sum(jnp.where(lanes == lane, cs, 0))
            c_at = jnp.sum(jnp.where(lanes == lane, c, 0))
            row2 = jnp.where(hit, tlane * ROWS_PER_TILE + j * L + lane, row)
            k2 = jnp.where(hit, rem - (cs_at - c_at), k)
            rem2 = jnp.where((found == 0) & jnp.logical_not(hit), rem - tot_j, rem)
            return (jnp.where(hit, 1, found), row2, k2, rem2)

        _, row, k, _ = lax.fori_loop(0, ROWS_PER_TILE // L, loc_row,
                                     (jnp.int32(0), jnp.int32(0), jnp.int32(0),
                                      rem0))

        pltpu.sync_copy(grid_in.at[row], rowb)

        def loc_col(j, carry):
            found, col, rem = carry
            c = jnp.where(rowb[pl.ds(j * L, L)] == ORCHARD, 1, 0)
            tot_j = jnp.sum(c)
            cs = jnp.cumsum(c)
            hit = (found == 0) & (tot_j > rem)
            lane = jnp.min(jnp.where(cs > rem, lanes, L))
            col2 = jnp.where(hit, j * L + lane, col)
            rem2 = jnp.where((found == 0) & jnp.logical_not(hit), rem - tot_j, rem)
            return (jnp.where(hit, 1, found), col2, rem2)

        _, col, _ = lax.fori_loop(0, G // L, loc_col,
                                  (jnp.int32(0), jnp.int32(0), k))

        spawn_on = sf & (tot > 0)
        spawnP = row * G + col

        pos = sv
        x = pos >> 10
        y = pos & (G - 1)
        a = jnp.clip(mov, 0, 4)
        dx = jnp.where(a == 1, -1, 0) + jnp.where(a == 2, 1, 0)
        dy = jnp.where(a == 3, -1, 0) + jnp.where(a == 4, 1, 0)
        nx = jnp.clip(x + dx, 0, G - 1)
        ny = jnp.clip(y + dy, 0, G - 1)
        P = nx * G + ny
        p_v[...] = P

        nx0 = nx[0]
        srow = jnp.where(spawn_on, row, nx0)
        rid = jnp.where(lanes < NA, nx, jnp.where(lanes == NA, srow, nx0))
        rid_v[...] = rid
        pltpu.async_copy(grid_in.at[rid_v], rows16, sem2).wait()
        O = plsc.load_gather(rows16, [lanes, ny])
        base_vals = jnp.where(spawn_on & (P == spawnP), APPLE, O)

        ate_vec = jnp.zeros(L, jnp.int32)
        ate_s = [None] * NA
        for i in reversed(range(NA)):
            prior = jnp.sum(jnp.where((ate_vec > 0) & (P == P[i]), 1, 0))
            cur = jnp.where(prior > 0, ORCHARD, base_vals[i])
            ate_i = (eatv[i] >= 1) & (cur == APPLE)
            ate_s[i] = ate_i
            ate_vec = jnp.where((lanes == i) & ate_i, 1, ate_vec)

        eaten_at = jnp.zeros(L, jnp.int32)
        for j in range(NA):
            eaten_at = eaten_at | jnp.where((P == P[j]) & ate_s[j], 1, 0)

        vfin = jnp.where(eaten_at > 0, ORCHARD, base_vals)
        item = jnp.where(nx == 1, 0.0, jnp.where(vfin == APPLE, 1.0, 2.0))
        obs_v[pl.ds(0, L)] = P.astype(jnp.float32)
        obs_v[pl.ds(L, L)] = item
        obs_v[pl.ds(2 * L, L)] = ate_vec.astype(jnp.float32)
        obs_v[pl.ds(3 * L, L)] = jnp.zeros(L, jnp.float32)
        pltpu.sync_copy(obs_v, obs_out)
        pltpu.sync_copy(p_v, pos_out)

        ate_b = ate_vec > 0
        ones_f = jnp.full(L, APPLE, jnp.float32)
        twos_f = jnp.full(L, ORCHARD, jnp.float32)
        for s in range(L):
            rs = rid[s]
            s_vec = jnp.full(L, s, jnp.int32)
            plsc.store_scatter(rows16, [s_vec, jnp.full(L, col, jnp.int32)],
                               ones_f,
                               mask=(lanes == 0) & spawn_on & (row == rs))
            plsc.store_scatter(rows16, [s_vec, ny], twos_f,
                               mask=(lanes < NA) & ate_b & (nx == rs))

        pltpu.async_copy(rows16, grid_out.at[rid_v], sem2).wait()


@functools.partial(jax.jit, static_argnames=())
def kernel(grid, agent_position, actions, rng_key):
    kd = lax.bitcast_convert_type(
        jr.key_data(rng_key).astype(jnp.uint32), jnp.int32)
    scal = jnp.concatenate([
        agent_position[0].astype(jnp.int32), kd, jnp.zeros(6, jnp.int32),
    ])
    act = actions.astype(jnp.int32).reshape(NA * 2)

    mesh = plsc.VectorSubcoreMesh(core_axis_name="c", subcore_axis_name="s",
                                  num_cores=1, num_subcores=16)
    grid_out, obs, poso = pl.kernel(
        _body,
        out_type=[
            jax.ShapeDtypeStruct((G, G), jnp.float32),
            jax.ShapeDtypeStruct((4 * L,), jnp.float32),
            jax.ShapeDtypeStruct((L,), jnp.int32),
        ],
        mesh=mesh,
        compiler_params=pltpu.CompilerParams(needs_layout_passes=False),
        scratch_types=[
            pltpu.VMEM((ROWS_PER_TILE, G), jnp.float32),
            pltpu.VMEM((ROWS_PER_TILE,), jnp.int32),
            pltpu.VMEM((L,), jnp.int32),
            pltpu.VMEM_SHARED((G,), jnp.int32),
            pltpu.VMEM_SHARED((L * L,), jnp.int32),
            pltpu.VMEM((G,), jnp.int32),
            pltpu.VMEM((L * L,), jnp.int32),
            pltpu.VMEM((G,), jnp.float32),
            pltpu.VMEM((L,), jnp.int32),
            pltpu.VMEM((L,), jnp.int32),
            pltpu.VMEM((L, G), jnp.float32),
            pltpu.VMEM((L,), jnp.int32),
            pltpu.VMEM((L,), jnp.int32),
            pltpu.VMEM((4 * L,), jnp.float32),
            pltpu.SemaphoreType.DMA,
            pltpu.SemaphoreType.DMA,
            pltpu.SemaphoreType.DMA,
            pltpu.SemaphoreType.DMA,
            pltpu.SemaphoreType.DMA,
            pltpu.SemaphoreType.DMA,
        ],
    )(grid[0], scal, act)

    loc_obs = obs[0:NA].reshape(NA, 1)
    item_obs = obs[L:L + NA].reshape(NA, 1)
    rew_obs = obs[2 * L:2 * L + NA].reshape(NA, 1)
    return (loc_obs, item_obs, rew_obs, grid_out[None],
            poso[0:NA].reshape(1, NA))
